# baseline (speedup 1.0000x reference)
import jax
import jax.numpy as jnp
from jax.experimental import pallas as pl
from jax.experimental.pallas import tpu as pltpu

B = 2
S0 = 256
S1 = 256
N = S0 + S1
F = 128
BN = B * N
NEG_SLOPE = 0.2


def _mm(a, b):
    return jax.lax.dot_general(
        a, b, (((1,), (0,)), ((), ())),
        preferred_element_type=jnp.float32)


def _mm_tt(a, b):
    return jax.lax.dot_general(
        a, b, (((0,), (0,)), ((), ())),
        preferred_element_type=jnp.float32)


def _col(v):
    return jnp.reshape(v, (F, 1))


def _layer_t(xt, W, q, k, cb, lw, lb, gam, bet, diag_neg):
    wa, wb = W[0], W[1]
    q, k = _col(q), _col(k)
    cb, lb, gam, bet = _col(cb), _col(lb), _col(gam), _col(bet)
    xw = (_mm_tt(wa, xt), _mm_tt(wb, xt))
    qv = (_mm_tt(q, xw[0]), _mm_tt(q, xw[1]))
    kv = (_mm_tt(xw[0], k), _mm_tt(xw[1], k))

    aggr_parts = []
    for b in range(B):
        for g in range(2):
            d0 = b * N + g * S0
            pre = []
            for h in range(2):
                t = 0 if g == h else 1
                s0 = b * N + h * S0
                p = qv[t][:, d0:d0 + S0] + kv[t][s0:s0 + S0]
                p = jnp.maximum(p, NEG_SLOPE * p)
                if h == g:
                    p = p + diag_neg
                pre.append(p)
            amax = jnp.maximum(
                jnp.max(pre[0], axis=0, keepdims=True),
                jnp.max(pre[1], axis=0, keepdims=True))
            acc = None
            den = None
            for h in range(2):
                t = 0 if g == h else 1
                s0 = b * N + h * S0
                e = jnp.exp(pre[h] - amax)
                dsum = jnp.sum(e, axis=0, keepdims=True)
                a = _mm(xw[t][:, s0:s0 + S0], e)
                den = dsum if den is None else den + dsum
                acc = a if acc is None else acc + a
            aggr_parts.append(acc / (den + 1e-16))
    aggr = jnp.concatenate(aggr_parts, axis=1)

    msg1 = jnp.maximum(aggr + cb, 0.0)
    msg2 = _mm(lw[:, :F], xt) + _mm(lw[:, F:], msg1) + lb
    mu = jnp.mean(msg2, axis=1, keepdims=True)
    var = jnp.mean((msg2 - mu) * (msg2 - mu), axis=1, keepdims=True)
    msg3 = (msg2 - mu) / jnp.sqrt(var + 1e-5) * gam + bet
    return xt + msg3


_L2_SHAPES = (((2, F, F), jnp.float32), ((F,), jnp.float32),
              ((F,), jnp.float32), ((F,), jnp.float32),
              ((F, 2 * F), jnp.float32), ((F,), jnp.float32),
              ((F,), jnp.float32), ((F,), jnp.float32))


def _rgat_body(d0_ref, d1_ref,
               W0, q0, k0, cb0, lw0, lb0, gam0, bet0,
               W1h, q1h, k1h, cb1h, lw1h, lb1h, gam1h, bet1h,
               o0_ref, o1_ref,
               W1, q1, k1, cb1, lw1, lb1, gam1, bet1, sems):
    l2h = (W1h, q1h, k1h, cb1h, lw1h, lb1h, gam1h, bet1h)
    l2v = (W1, q1, k1, cb1, lw1, lb1, gam1, bet1)
    copies = [pltpu.make_async_copy(h, v, sems.at[i])
              for i, (h, v) in enumerate(zip(l2h, l2v))]
    for c in copies:
        c.start()

    di = jax.lax.broadcasted_iota(jnp.int32, (S0, S0), 0)
    si = jax.lax.broadcasted_iota(jnp.int32, (S0, S0), 1)
    diag_neg = jnp.where(di == si, -1e30, 0.0)

    xt = jnp.concatenate([d0_ref[0], d1_ref[0], d0_ref[1], d1_ref[1]], axis=1)
    xt = _layer_t(xt, W0[...], q0[...], k0[...], cb0[...], lw0[...],
                  lb0[...], gam0[...], bet0[...], diag_neg)
    for c in copies:
        c.wait()
    xt = _layer_t(xt, W1[...], q1[...], k1[...], cb1[...], lw1[...],
                  lb1[...], gam1[...], bet1[...], diag_neg)
    for b in range(B):
        o0_ref[b] = xt[:, b * N:b * N + S0]
        o1_ref[b] = xt[:, b * N + S0:(b + 1) * N]


def kernel(desc0, desc1, W0, q0, k0, cb0, lw0, lb0, gam0, bet0,
           W1, q1, k1, cb1, lw1, lb1, gam1, bet1):
    return pl.pallas_call(
        _rgat_body,
        in_specs=[pl.BlockSpec(memory_space=pltpu.MemorySpace.VMEM)] * 10
                 + [pl.BlockSpec(memory_space=pl.ANY)] * 8,
        out_shape=(jax.ShapeDtypeStruct((B, F, S0), jnp.float32),
                   jax.ShapeDtypeStruct((B, F, S1), jnp.float32)),
        scratch_shapes=[pltpu.VMEM(s, d) for s, d in _L2_SHAPES]
                       + [pltpu.SemaphoreType.DMA((8,))],
    )(desc0, desc1,
      W0, q0.reshape(F), k0.reshape(F), cb0, lw0, lb0, gam0, bet0,
      W1, q1.reshape(F), k1.reshape(F), cb1, lw1, lb1, gam1, bet1)


# speedup vs baseline: 1.0210x; 1.0210x over previous
import jax
import jax.numpy as jnp
from jax.experimental import pallas as pl

B = 2
S0 = 256
S1 = 256
N = S0 + S1
F = 128
BN = B * N
NEG_SLOPE = 0.2


def _mm(a, b):
    return jax.lax.dot_general(
        a, b, (((1,), (0,)), ((), ())),
        preferred_element_type=jnp.float32)


def _mm_tt(a, b):
    return jax.lax.dot_general(
        a, b, (((0,), (0,)), ((), ())),
        preferred_element_type=jnp.float32)


def _col(v):
    return jnp.reshape(v, (F, 1))


def _layer_t(xt, W, q, k, cb, lw, lb, gam, bet, diag_neg):
    wa, wb = W[0], W[1]
    q, k = _col(q), _col(k)
    cb, lb, gam, bet = _col(cb), _col(lb), _col(gam), _col(bet)
    xw = (_mm_tt(wa, xt), _mm_tt(wb, xt))
    qv = (_mm_tt(q, xw[0]), _mm_tt(q, xw[1]))
    kv = (_mm_tt(xw[0], k), _mm_tt(xw[1], k))

    aggr_parts = []
    for b in range(B):
        for g in range(2):
            d0 = b * N + g * S0
            pre = []
            for h in range(2):
                t = 0 if g == h else 1
                s0 = b * N + h * S0
                p = qv[t][:, d0:d0 + S0] + kv[t][s0:s0 + S0]
                p = jnp.maximum(p, NEG_SLOPE * p)
                if h == g:
                    p = p + diag_neg
                pre.append(p)
            amax = jnp.maximum(
                jnp.max(pre[0], axis=0, keepdims=True),
                jnp.max(pre[1], axis=0, keepdims=True))
            acc = None
            den = None
            for h in range(2):
                t = 0 if g == h else 1
                s0 = b * N + h * S0
                e = jnp.exp(pre[h] - amax)
                dsum = jnp.sum(e, axis=0, keepdims=True)
                a = _mm(xw[t][:, s0:s0 + S0], e)
                den = dsum if den is None else den + dsum
                acc = a if acc is None else acc + a
            aggr_parts.append(acc / (den + 1e-16))
    aggr = jnp.concatenate(aggr_parts, axis=1)

    msg1 = jnp.maximum(aggr + cb, 0.0)
    msg2 = _mm(lw[:, :F], xt) + _mm(lw[:, F:], msg1) + lb
    mu = jnp.mean(msg2, axis=1, keepdims=True)
    var = jnp.mean((msg2 - mu) * (msg2 - mu), axis=1, keepdims=True)
    msg3 = (msg2 - mu) / jnp.sqrt(var + 1e-5) * gam + bet
    return xt + msg3


def _rgat_body(d0_ref, d1_ref,
               W0, q0, k0, cb0, lw0, lb0, gam0, bet0,
               W1, q1, k1, cb1, lw1, lb1, gam1, bet1,
               o0_ref, o1_ref):
    di = jax.lax.broadcasted_iota(jnp.int32, (S0, S0), 0)
    si = jax.lax.broadcasted_iota(jnp.int32, (S0, S0), 1)
    diag_neg = jnp.where(di == si, -1e30, 0.0)

    xt = jnp.concatenate([d0_ref[0], d1_ref[0], d0_ref[1], d1_ref[1]], axis=1)
    xt = _layer_t(xt, W0[...], q0[...], k0[...], cb0[...], lw0[...],
                  lb0[...], gam0[...], bet0[...], diag_neg)
    xt = _layer_t(xt, W1[...], q1[...], k1[...], cb1[...], lw1[...],
                  lb1[...], gam1[...], bet1[...], diag_neg)
    for b in range(B):
        o0_ref[b] = xt[:, b * N:b * N + S0]
        o1_ref[b] = xt[:, b * N + S0:(b + 1) * N]


def kernel(desc0, desc1, W0, q0, k0, cb0, lw0, lb0, gam0, bet0,
           W1, q1, k1, cb1, lw1, lb1, gam1, bet1):
    return pl.pallas_call(
        _rgat_body,
        out_shape=(jax.ShapeDtypeStruct((B, F, S0), jnp.float32),
                   jax.ShapeDtypeStruct((B, F, S1), jnp.float32)),
    )(desc0, desc1,
      W0, q0.reshape(F), k0.reshape(F), cb0, lw0, lb0, gam0, bet0,
      W1, q1.reshape(F), k1.reshape(F), cb1, lw1, lb1, gam1, bet1)
